# baseline (speedup 1.0000x reference)
import functools

import jax
import jax.numpy as jnp
from jax import lax
from jax.experimental import pallas as pl
from jax.experimental.pallas import tpu as pltpu
from jax.experimental.pallas import tpu_sc as plsc

B, A, T, D, H = 32, 12, 80, 128, 8
E_RAW = 132
EP = 144
NG = EP // 16
N = B * A
F = B * T
M = F * A
HD = H * D
R_ROWS = N * T
BLK = 2048
NBLK = M // BLK
FB = 64
NFB = F // FB
NW = 32
FPW = F // NW
XLW = (A + 1) * HD


def _mlp1_body(hexid_ref, pid_ref, th_ref, tp_ref, ti_ref, b1_ref,
               out_ref, stats_ref):
    i = pl.program_id(0)
    rows = hexid_ref.shape[0]
    hexid = hexid_ref[...]
    pid = pid_ref[...]
    ioh = lax.broadcasted_iota(jnp.int32, (rows, 600), 1)
    oh_h = (ioh == hexid).astype(jnp.float32)
    oh_p = (ioh == pid).astype(jnp.float32)
    grow = i * rows + lax.broadcasted_iota(jnp.int32, (rows, 1), 0)
    aid = (grow // T) % A
    oh_a = (lax.broadcasted_iota(jnp.int32, (rows, A), 1) == aid
            ).astype(jnp.float32)
    y = (jnp.dot(oh_h, th_ref[...], preferred_element_type=jnp.float32)
         + jnp.dot(oh_p, tp_ref[...], preferred_element_type=jnp.float32)
         + jnp.dot(oh_a, ti_ref[...], preferred_element_type=jnp.float32)
         + b1_ref[...])
    out_ref[...] = y

    @pl.when(i == 0)
    def _():
        stats_ref[...] = jnp.zeros_like(stats_ref)

    stats_ref[0:1, :] += jnp.sum(y, axis=0, keepdims=True)
    stats_ref[1:2, :] += jnp.sum(y * y, axis=0, keepdims=True)


def _mlp_body(y_ref, st_ref, g_ref, be_ref, w_ref, b_ref, out_ref, stats_ref):
    i = pl.program_id(0)
    mu = st_ref[0:1, :] / R_ROWS
    ms = st_ref[1:2, :] / R_ROWS
    inv = lax.rsqrt(ms - mu * mu + 1e-5)
    h = (y_ref[...] - mu) * inv * g_ref[...] + be_ref[...]
    h = jnp.maximum(h, 0.0)
    y = jnp.dot(h, w_ref[...], preferred_element_type=jnp.float32) + b_ref[...]
    out_ref[...] = y

    @pl.when(i == 0)
    def _():
        stats_ref[...] = jnp.zeros_like(stats_ref)

    stats_ref[0:1, :] += jnp.sum(y, axis=0, keepdims=True)
    stats_ref[1:2, :] += jnp.sum(y * y, axis=0, keepdims=True)


def _bnfin_body(y_ref, st_ref, g_ref, be_ref, out_ref):
    mu = st_ref[0:1, :] / R_ROWS
    ms = st_ref[1:2, :] / R_ROWS
    inv = lax.rsqrt(ms - mu * mu + 1e-5)
    h = (y_ref[...] - mu) * inv * g_ref[...] + be_ref[...]
    out_ref[...] = jnp.maximum(h, 0.0)


def _proj_body(x_ref, wl_ref, bl_ref, wr_ref, br_ref, xl_ref, xr_ref):
    x = x_ref[...]
    xl_ref[...] = jnp.dot(x, wl_ref[...],
                          preferred_element_type=jnp.float32) + bl_ref[...]
    xr_ref[...] = jnp.dot(x, wr_ref[...],
                          preferred_element_type=jnp.float32) + br_ref[...]


def _agg_body(wm_ref, xl_ref, xres_ref, bias_ref, g_ref, b_ref, out_ref):
    wm = wm_ref[...]
    xlb = xl_ref[...]
    for j in range(A):
        acc = jnp.zeros((FB, D), jnp.float32)
        for h in range(H):
            for i in range(A):
                w = wm[:, h * 256 + j * 16 + i][:, None]
                xs = xlb[:, i * HD + h * D:(i * HD + h * D) + D]
                acc = acc + w * xs
        v = acc * (1.0 / H) + bias_ref[...]
        v = v + xres_ref[:, j * D:(j + 1) * D]
        mu = jnp.mean(v, axis=-1, keepdims=True)
        var = jnp.mean((v - mu) * (v - mu), axis=-1, keepdims=True)
        v = (v - mu) * lax.rsqrt(var + 1e-5) * g_ref[...] + b_ref[...]
        out_ref[:, j * D:(j + 1) * D] = v


def _sc_body(xl_hbm, xr_hbm, ed_hbm, ea_hbm, wea_hbm, wout_hbm,
             xl_v, xr_v, ed_v, ea_v, wea_v, w_v, den_v):
    wid = lax.axis_index("s") * 2 + lax.axis_index("c")
    zi = jnp.zeros((16,), jnp.int32)
    zf = jnp.zeros((16,), jnp.float32)

    pltpu.sync_copy(wea_hbm, wea_v)
    for k in range(HD // 16):
        xl_v[pl.ds(A * HD + k * 16, 16)] = zf
        xr_v[pl.ds(A * HD + k * 16, 16)] = zf

    def frame_body(k, carry):
        f = wid * FPW + k
        pltpu.sync_copy(xl_hbm.at[pl.ds(f * (A * HD), A * HD)],
                        xl_v.at[pl.ds(0, A * HD)])
        pltpu.sync_copy(xr_hbm.at[pl.ds(f * (A * HD), A * HD)],
                        xr_v.at[pl.ds(0, A * HD)])
        pltpu.sync_copy(ed_hbm.at[f], ed_v)
        pltpu.sync_copy(ea_hbm.at[f], ea_v)

        def zero_body(q, c):
            w_v[pl.ds(q * 16, 16)] = zf
            return c

        lax.fori_loop(0, 128, zero_body, 0)
        for q in range(8):
            den_v[pl.ds(q * 16, 16)] = zf

        sb = [ed_v[pl.ds(g * 16, 16)] for g in range(NG)]
        db = [ed_v[pl.ds(EP + g * 16, 16)] for g in range(NG)]
        wx = [ed_v[pl.ds(2 * EP + g * 16, 16)] for g in range(NG)]
        dn = [ed_v[pl.ds(3 * EP + g * 16, 16)] for g in range(NG)]
        at = [ea_v[pl.ds(g * 16, 16)] for g in range(NG)]

        for h in range(H):
            hbase = h * D

            def dbody(d, accs):
                offv = zi + (hbase + d)
                we = plsc.load_gather(wea_v, [offv])
                aw = plsc.load_gather(wea_v, [offv + HD])
                out = []
                for g in range(NG):
                    gl = plsc.load_gather(xl_v, [sb[g] + offv])
                    gr = plsc.load_gather(xr_v, [db[g] + offv])
                    z = gl + gr + at[g] * we
                    z = jnp.maximum(z, 0.2 * z)
                    out.append(accs[g] + z * aw)
                return tuple(out)

            accs = plsc.parallel_loop(
                0, D, 1, unroll=4, carry=tuple(zf for _ in range(NG)))(dbody)

            m = accs[0]
            for g in range(1, NG):
                m = jnp.maximum(m, accs[g])
            mx = jnp.max(m)
            exs = [jnp.exp(a - mx) for a in accs]
            for g in range(NG):
                plsc.addupdate_scatter(den_v, [dn[g] + (h * 16)], exs[g])
            for g in range(NG):
                dsum = plsc.load_gather(den_v, [dn[g] + (h * 16)])
                alpha = exs[g] / (dsum + 1e-16)
                plsc.addupdate_scatter(w_v, [wx[g] + (h * 256)], alpha)

        pltpu.sync_copy(w_v, wout_hbm.at[f])
        return carry

    lax.fori_loop(0, FPW, frame_body, 0)


def _sc_call(xl_flat, xr_flat, edata, eattr, wea):
    mesh = plsc.VectorSubcoreMesh(core_axis_name="c", subcore_axis_name="s",
                                  num_cores=2, num_subcores=16)
    fn = pl.kernel(
        _sc_body,
        out_type=jax.ShapeDtypeStruct((F, 2048), jnp.float32),
        mesh=mesh,
        scratch_types=[
            pltpu.VMEM((XLW,), jnp.float32),
            pltpu.VMEM((XLW,), jnp.float32),
            pltpu.VMEM((4 * EP,), jnp.int32),
            pltpu.VMEM((EP,), jnp.float32),
            pltpu.VMEM((2 * HD,), jnp.float32),
            pltpu.VMEM((2048,), jnp.float32),
            pltpu.VMEM((128,), jnp.float32),
        ],
        compiler_params=pltpu.CompilerParams(needs_layout_passes=False),
    )
    return fn(xl_flat, xr_flat, edata, eattr, wea)


def _stats_spec():
    return pl.BlockSpec((8, None), lambda i: (0, 0))


def kernel(state_feat, padding_mask, agent_ids, edge_index, edge_attr, params):
    p = params
    f32 = jnp.float32

    hex_id = state_feat[..., -1]
    hex_id = jnp.where(hex_id == 1e9, 598.0, hex_id).astype(jnp.int32)
    hexid = hex_id.reshape(R_ROWS, 1)
    pid = jnp.broadcast_to(jnp.clip(agent_ids, 0)[:, None],
                           (N, T)).reshape(R_ROWS, 1).astype(jnp.int32)
    th = p['hex_table'] @ p['W1'][0:12]
    tp = p['player_table'] @ p['W1'][12:24]
    ti = p['index_table'] @ p['W1'][24:36]
    b1 = p['b1'][None, :]

    grid1 = pl.pallas_call(
        _mlp1_body,
        grid=(NBLK,),
        in_specs=[
            pl.BlockSpec((BLK, 1), lambda i: (i, 0)),
            pl.BlockSpec((BLK, 1), lambda i: (i, 0)),
            pl.BlockSpec((600, 64), lambda i: (0, 0)),
            pl.BlockSpec((600, 64), lambda i: (0, 0)),
            pl.BlockSpec((A, 64), lambda i: (0, 0)),
            pl.BlockSpec((1, 64), lambda i: (0, 0)),
        ],
        out_specs=[
            pl.BlockSpec((BLK, 64), lambda i: (i, 0)),
            pl.BlockSpec((8, 64), lambda i: (0, 0)),
        ],
        out_shape=[
            jax.ShapeDtypeStruct((R_ROWS, 64), f32),
            jax.ShapeDtypeStruct((8, 64), f32),
        ],
    )
    y1, s1 = grid1(hexid, pid, th, tp, ti, b1)

    def mlp_layer(y, st, g, be, w, b, fout):
        fin = y.shape[1]
        call = pl.pallas_call(
            _mlp_body,
            grid=(NBLK,),
            in_specs=[
                pl.BlockSpec((BLK, fin), lambda i: (i, 0)),
                pl.BlockSpec((8, fin), lambda i: (0, 0)),
                pl.BlockSpec((1, fin), lambda i: (0, 0)),
                pl.BlockSpec((1, fin), lambda i: (0, 0)),
                pl.BlockSpec((fin, fout), lambda i: (0, 0)),
                pl.BlockSpec((1, fout), lambda i: (0, 0)),
            ],
            out_specs=[
                pl.BlockSpec((BLK, fout), lambda i: (i, 0)),
                pl.BlockSpec((8, fout), lambda i: (0, 0)),
            ],
            out_shape=[
                jax.ShapeDtypeStruct((R_ROWS, fout), f32),
                jax.ShapeDtypeStruct((8, fout), f32),
            ],
        )
        return call(y, st, g[None, :], be[None, :], w, b[None, :])

    y2, s2 = mlp_layer(y1, s1, p['g1'], p['be1'], p['W2'], p['b2'], 256)
    y3, s3 = mlp_layer(y2, s2, p['g2'], p['be2'], p['W3'], p['b3'], D)

    y3n = y3.reshape(B, A, T, D).transpose(0, 2, 1, 3).reshape(M, D)
    xn = pl.pallas_call(
        _bnfin_body,
        grid=(NBLK,),
        in_specs=[
            pl.BlockSpec((BLK, D), lambda i: (i, 0)),
            pl.BlockSpec((8, D), lambda i: (0, 0)),
            pl.BlockSpec((1, D), lambda i: (0, 0)),
            pl.BlockSpec((1, D), lambda i: (0, 0)),
        ],
        out_specs=pl.BlockSpec((BLK, D), lambda i: (i, 0)),
        out_shape=jax.ShapeDtypeStruct((M, D), f32),
    )(y3n, s3, p['g3'][None, :], p['be3'][None, :])

    src = edge_index[:, :, 0, :].reshape(F, E_RAW).astype(jnp.int32)
    dst = edge_index[:, :, 1, :].reshape(F, E_RAW).astype(jnp.int32)
    padi = jnp.full((F, EP - E_RAW), A, jnp.int32)
    src = jnp.concatenate([src, padi], axis=1)
    dst = jnp.concatenate([dst, padi], axis=1)
    attr = jnp.concatenate(
        [edge_attr.reshape(F, E_RAW),
         jnp.zeros((F, EP - E_RAW), f32)], axis=1)
    edata = jnp.concatenate(
        [src * HD, dst * HD, dst * 16 + src, dst], axis=1)

    for li in range(3):
        wl, bl = p[f'gat{li}_Wl'], p[f'gat{li}_bl']
        wr, br = p[f'gat{li}_Wr'], p[f'gat{li}_br']
        xl, xr = pl.pallas_call(
            _proj_body,
            grid=(NBLK,),
            in_specs=[
                pl.BlockSpec((BLK, D), lambda i: (i, 0)),
                pl.BlockSpec((D, HD), lambda i: (0, 0)),
                pl.BlockSpec((1, HD), lambda i: (0, 0)),
                pl.BlockSpec((D, HD), lambda i: (0, 0)),
                pl.BlockSpec((1, HD), lambda i: (0, 0)),
            ],
            out_specs=[
                pl.BlockSpec((BLK, HD), lambda i: (i, 0)),
                pl.BlockSpec((BLK, HD), lambda i: (i, 0)),
            ],
            out_shape=[
                jax.ShapeDtypeStruct((M, HD), f32),
                jax.ShapeDtypeStruct((M, HD), f32),
            ],
        )(xn, wl, bl[None, :], wr, br[None, :])

        wea = jnp.concatenate([p[f'gat{li}_We'].reshape(HD),
                               p[f'gat{li}_att'].reshape(HD)])
        wm = _sc_call(xl.reshape(M * HD), xr.reshape(M * HD), edata, attr, wea)

        xn = pl.pallas_call(
            _agg_body,
            grid=(NFB,),
            in_specs=[
                pl.BlockSpec((FB, 2048), lambda i: (i, 0)),
                pl.BlockSpec((FB, A * HD), lambda i: (i, 0)),
                pl.BlockSpec((FB, A * D), lambda i: (i, 0)),
                pl.BlockSpec((1, D), lambda i: (0, 0)),
                pl.BlockSpec((1, D), lambda i: (0, 0)),
                pl.BlockSpec((1, D), lambda i: (0, 0)),
            ],
            out_specs=pl.BlockSpec((FB, A * D), lambda i: (i, 0)),
            out_shape=jax.ShapeDtypeStruct((F, A * D), f32),
        )(wm, xl.reshape(F, A * HD), xn.reshape(F, A * D),
          p[f'gat{li}_bias'][None, :], p[f'ln{li}_g'][None, :],
          p[f'ln{li}_b'][None, :]).reshape(M, D)

    return xn.reshape(B, T, A, D).transpose(0, 2, 1, 3).reshape(N, T, D)


# speedup vs baseline: 2.5666x; 2.5666x over previous
import functools

import jax
import jax.numpy as jnp
from jax import lax
from jax.experimental import pallas as pl
from jax.experimental.pallas import tpu as pltpu
from jax.experimental.pallas import tpu_sc as plsc

B, A, T, D, H = 32, 12, 80, 128, 8
E_RAW = 132
EP = 144
NG = EP // 16
N = B * A
F = B * T
M = F * A
HD = H * D
R_ROWS = N * T
BLK = 2048
NBLK = M // BLK
FB = 64
NFB = F // FB
NW = 32
FPW = F // NW
HDP = HD + 8
XLW = (A + 1) * HDP


def _mlp1_body(hexid_ref, pid_ref, th_ref, tp_ref, ti_ref, b1_ref,
               out_ref, stats_ref):
    i = pl.program_id(0)
    rows = hexid_ref.shape[0]
    hexid = hexid_ref[...]
    pid = pid_ref[...]
    ioh = lax.broadcasted_iota(jnp.int32, (rows, 600), 1)
    oh_h = (ioh == hexid).astype(jnp.float32)
    oh_p = (ioh == pid).astype(jnp.float32)
    grow = i * rows + lax.broadcasted_iota(jnp.int32, (rows, 1), 0)
    aid = (grow // T) % A
    oh_a = (lax.broadcasted_iota(jnp.int32, (rows, A), 1) == aid
            ).astype(jnp.float32)
    y = (jnp.dot(oh_h, th_ref[...], preferred_element_type=jnp.float32)
         + jnp.dot(oh_p, tp_ref[...], preferred_element_type=jnp.float32)
         + jnp.dot(oh_a, ti_ref[...], preferred_element_type=jnp.float32)
         + b1_ref[...])
    out_ref[...] = y

    @pl.when(i == 0)
    def _():
        stats_ref[...] = jnp.zeros_like(stats_ref)

    stats_ref[0:1, :] += jnp.sum(y, axis=0, keepdims=True)
    stats_ref[1:2, :] += jnp.sum(y * y, axis=0, keepdims=True)


def _mlp_body(y_ref, st_ref, g_ref, be_ref, w_ref, b_ref, out_ref, stats_ref):
    i = pl.program_id(0)
    mu = st_ref[0:1, :] / R_ROWS
    ms = st_ref[1:2, :] / R_ROWS
    inv = lax.rsqrt(ms - mu * mu + 1e-5)
    h = (y_ref[...] - mu) * inv * g_ref[...] + be_ref[...]
    h = jnp.maximum(h, 0.0)
    y = jnp.dot(h, w_ref[...], preferred_element_type=jnp.float32) + b_ref[...]
    out_ref[...] = y

    @pl.when(i == 0)
    def _():
        stats_ref[...] = jnp.zeros_like(stats_ref)

    stats_ref[0:1, :] += jnp.sum(y, axis=0, keepdims=True)
    stats_ref[1:2, :] += jnp.sum(y * y, axis=0, keepdims=True)


def _bnfin_body(y_ref, st_ref, g_ref, be_ref, out_ref):
    mu = st_ref[0:1, :] / R_ROWS
    ms = st_ref[1:2, :] / R_ROWS
    inv = lax.rsqrt(ms - mu * mu + 1e-5)
    h = (y_ref[...] - mu) * inv * g_ref[...] + be_ref[...]
    out_ref[...] = jnp.maximum(h, 0.0)


def _proj_body(x_ref, wl_ref, bl_ref, wr_ref, br_ref, xl_ref, xr_ref):
    x = x_ref[...]
    xl_ref[...] = jnp.dot(x, wl_ref[...],
                          preferred_element_type=jnp.float32) + bl_ref[...]
    xr_ref[...] = jnp.dot(x, wr_ref[...],
                          preferred_element_type=jnp.float32) + br_ref[...]


def _agg_body(wm_ref, xl_ref, xres_ref, bias_ref, g_ref, b_ref, out_ref):
    wm = wm_ref[...]
    xlb = xl_ref[...]
    for j in range(A):
        acc = jnp.zeros((FB, D), jnp.float32)
        for h in range(H):
            for i in range(A):
                w = wm[:, h * 256 + j * 16 + i][:, None]
                xs = xlb[:, i * HD + h * D:(i * HD + h * D) + D]
                acc = acc + w * xs
        v = acc * (1.0 / H) + bias_ref[...]
        v = v + xres_ref[:, j * D:(j + 1) * D]
        mu = jnp.mean(v, axis=-1, keepdims=True)
        var = jnp.mean((v - mu) * (v - mu), axis=-1, keepdims=True)
        v = (v - mu) * lax.rsqrt(var + 1e-5) * g_ref[...] + b_ref[...]
        out_ref[:, j * D:(j + 1) * D] = v


def _sc_body(xl_hbm, xr_hbm, ed_hbm, ea_hbm, wea_hbm, wout_hbm,
             xl_v, xr_v, ed_v, ea_v, wea_v, w_v, den_v):
    wid = lax.axis_index("s") * 2 + lax.axis_index("c")
    zi = jnp.zeros((16,), jnp.int32)
    zf = jnp.zeros((16,), jnp.float32)

    pltpu.sync_copy(wea_hbm, wea_v)
    for k in range(HD // 16):
        xl_v[pl.ds(A * HDP + k * 16, 16)] = zf
        xr_v[pl.ds(A * HDP + k * 16, 16)] = zf

    def frame_body(k, carry):
        f = wid * FPW + k
        pltpu.sync_copy(xl_hbm.at[pl.ds(f * (A * HDP), A * HDP)],
                        xl_v.at[pl.ds(0, A * HDP)])
        pltpu.sync_copy(xr_hbm.at[pl.ds(f * (A * HDP), A * HDP)],
                        xr_v.at[pl.ds(0, A * HDP)])
        pltpu.sync_copy(ed_hbm.at[f], ed_v)
        pltpu.sync_copy(ea_hbm.at[f], ea_v)

        def zero_body(q, c):
            w_v[pl.ds(q * 16, 16)] = zf
            return c

        lax.fori_loop(0, 128, zero_body, 0)
        for q in range(8):
            den_v[pl.ds(q * 16, 16)] = zf

        sb = [ed_v[pl.ds(g * 16, 16)] for g in range(NG)]
        db = [ed_v[pl.ds(EP + g * 16, 16)] for g in range(NG)]
        wx = [ed_v[pl.ds(2 * EP + g * 16, 16)] for g in range(NG)]
        dn = [ed_v[pl.ds(3 * EP + g * 16, 16)] for g in range(NG)]
        at = [ea_v[pl.ds(g * 16, 16)] for g in range(NG)]

        for h in range(H):
            hbase = h * D

            def dbody(d, accs):
                offv = zi + (hbase + d)
                we = plsc.load_gather(wea_v, [offv])
                aw = plsc.load_gather(wea_v, [offv + HD])
                out = []
                for g in range(NG):
                    gl = plsc.load_gather(xl_v, [sb[g] + offv])
                    gr = plsc.load_gather(xr_v, [db[g] + offv])
                    z = gl + gr + at[g] * we
                    z = jnp.maximum(z, 0.2 * z)
                    out.append(accs[g] + z * aw)
                return tuple(out)

            accs = lax.fori_loop(0, D, dbody, tuple(zf for _ in range(NG)))

            m = accs[0]
            for g in range(1, NG):
                m = jnp.maximum(m, accs[g])
            mx = jnp.max(m)
            exs = [jnp.exp(a - mx) for a in accs]
            for g in range(NG):
                plsc.addupdate_scatter(den_v, [dn[g] + (h * 16)], exs[g])
            for g in range(NG):
                dsum = plsc.load_gather(den_v, [dn[g] + (h * 16)])
                alpha = exs[g] / (dsum + 1e-16)
                plsc.addupdate_scatter(w_v, [wx[g] + (h * 256)], alpha)

        pltpu.sync_copy(w_v, wout_hbm.at[f])
        return carry

    lax.fori_loop(0, FPW, frame_body, 0)


def _sc_call(xl_flat, xr_flat, edata, eattr, wea):
    mesh = plsc.VectorSubcoreMesh(core_axis_name="c", subcore_axis_name="s",
                                  num_cores=2, num_subcores=16)
    fn = pl.kernel(
        _sc_body,
        out_type=jax.ShapeDtypeStruct((F, 2048), jnp.float32),
        mesh=mesh,
        scratch_types=[
            pltpu.VMEM((XLW,), jnp.float32),
            pltpu.VMEM((XLW,), jnp.float32),
            pltpu.VMEM((4 * EP,), jnp.int32),
            pltpu.VMEM((EP,), jnp.float32),
            pltpu.VMEM((2 * HD,), jnp.float32),
            pltpu.VMEM((2048,), jnp.float32),
            pltpu.VMEM((128,), jnp.float32),
        ],
        compiler_params=pltpu.CompilerParams(needs_layout_passes=False),
    )
    return fn(xl_flat, xr_flat, edata, eattr, wea)


def _stats_spec():
    return pl.BlockSpec((8, None), lambda i: (0, 0))


def kernel(state_feat, padding_mask, agent_ids, edge_index, edge_attr, params):
    p = params
    f32 = jnp.float32

    hex_id = state_feat[..., -1]
    hex_id = jnp.where(hex_id == 1e9, 598.0, hex_id).astype(jnp.int32)
    hexid = hex_id.reshape(R_ROWS, 1)
    pid = jnp.broadcast_to(jnp.clip(agent_ids, 0)[:, None],
                           (N, T)).reshape(R_ROWS, 1).astype(jnp.int32)
    th = p['hex_table'] @ p['W1'][0:12]
    tp = p['player_table'] @ p['W1'][12:24]
    ti = p['index_table'] @ p['W1'][24:36]
    b1 = p['b1'][None, :]

    grid1 = pl.pallas_call(
        _mlp1_body,
        grid=(NBLK,),
        in_specs=[
            pl.BlockSpec((BLK, 1), lambda i: (i, 0)),
            pl.BlockSpec((BLK, 1), lambda i: (i, 0)),
            pl.BlockSpec((600, 64), lambda i: (0, 0)),
            pl.BlockSpec((600, 64), lambda i: (0, 0)),
            pl.BlockSpec((A, 64), lambda i: (0, 0)),
            pl.BlockSpec((1, 64), lambda i: (0, 0)),
        ],
        out_specs=[
            pl.BlockSpec((BLK, 64), lambda i: (i, 0)),
            pl.BlockSpec((8, 64), lambda i: (0, 0)),
        ],
        out_shape=[
            jax.ShapeDtypeStruct((R_ROWS, 64), f32),
            jax.ShapeDtypeStruct((8, 64), f32),
        ],
    )
    y1, s1 = grid1(hexid, pid, th, tp, ti, b1)

    def mlp_layer(y, st, g, be, w, b, fout):
        fin = y.shape[1]
        call = pl.pallas_call(
            _mlp_body,
            grid=(NBLK,),
            in_specs=[
                pl.BlockSpec((BLK, fin), lambda i: (i, 0)),
                pl.BlockSpec((8, fin), lambda i: (0, 0)),
                pl.BlockSpec((1, fin), lambda i: (0, 0)),
                pl.BlockSpec((1, fin), lambda i: (0, 0)),
                pl.BlockSpec((fin, fout), lambda i: (0, 0)),
                pl.BlockSpec((1, fout), lambda i: (0, 0)),
            ],
            out_specs=[
                pl.BlockSpec((BLK, fout), lambda i: (i, 0)),
                pl.BlockSpec((8, fout), lambda i: (0, 0)),
            ],
            out_shape=[
                jax.ShapeDtypeStruct((R_ROWS, fout), f32),
                jax.ShapeDtypeStruct((8, fout), f32),
            ],
        )
        return call(y, st, g[None, :], be[None, :], w, b[None, :])

    y2, s2 = mlp_layer(y1, s1, p['g1'], p['be1'], p['W2'], p['b2'], 256)
    y3, s3 = mlp_layer(y2, s2, p['g2'], p['be2'], p['W3'], p['b3'], D)

    y3n = y3.reshape(B, A, T, D).transpose(0, 2, 1, 3).reshape(M, D)
    xn = pl.pallas_call(
        _bnfin_body,
        grid=(NBLK,),
        in_specs=[
            pl.BlockSpec((BLK, D), lambda i: (i, 0)),
            pl.BlockSpec((8, D), lambda i: (0, 0)),
            pl.BlockSpec((1, D), lambda i: (0, 0)),
            pl.BlockSpec((1, D), lambda i: (0, 0)),
        ],
        out_specs=pl.BlockSpec((BLK, D), lambda i: (i, 0)),
        out_shape=jax.ShapeDtypeStruct((M, D), f32),
    )(y3n, s3, p['g3'][None, :], p['be3'][None, :])

    src = edge_index[:, :, 0, :].reshape(F, E_RAW).astype(jnp.int32)
    dst = edge_index[:, :, 1, :].reshape(F, E_RAW).astype(jnp.int32)
    padi = jnp.full((F, EP - E_RAW), A, jnp.int32)
    src = jnp.concatenate([src, padi], axis=1)
    dst = jnp.concatenate([dst, padi], axis=1)
    attr = jnp.concatenate(
        [edge_attr.reshape(F, E_RAW),
         jnp.zeros((F, EP - E_RAW), f32)], axis=1)
    edata = jnp.concatenate(
        [src * HDP, dst * HDP, dst * 16 + src, dst], axis=1)

    for li in range(3):
        wl, bl = p[f'gat{li}_Wl'], p[f'gat{li}_bl']
        wr, br = p[f'gat{li}_Wr'], p[f'gat{li}_br']
        xl, xr = pl.pallas_call(
            _proj_body,
            grid=(NBLK,),
            in_specs=[
                pl.BlockSpec((BLK, D), lambda i: (i, 0)),
                pl.BlockSpec((D, HD), lambda i: (0, 0)),
                pl.BlockSpec((1, HD), lambda i: (0, 0)),
                pl.BlockSpec((D, HD), lambda i: (0, 0)),
                pl.BlockSpec((1, HD), lambda i: (0, 0)),
            ],
            out_specs=[
                pl.BlockSpec((BLK, HD), lambda i: (i, 0)),
                pl.BlockSpec((BLK, HD), lambda i: (i, 0)),
            ],
            out_shape=[
                jax.ShapeDtypeStruct((M, HD), f32),
                jax.ShapeDtypeStruct((M, HD), f32),
            ],
        )(xn, wl, bl[None, :], wr, br[None, :])

        wea = jnp.concatenate([p[f'gat{li}_We'].reshape(HD),
                               p[f'gat{li}_att'].reshape(HD)])
        xlp = jnp.pad(xl, ((0, 0), (0, HDP - HD))).reshape(M * HDP)
        xrp = jnp.pad(xr, ((0, 0), (0, HDP - HD))).reshape(M * HDP)
        wm = _sc_call(xlp, xrp, edata, attr, wea)

        xn = pl.pallas_call(
            _agg_body,
            grid=(NFB,),
            in_specs=[
                pl.BlockSpec((FB, 2048), lambda i: (i, 0)),
                pl.BlockSpec((FB, A * HD), lambda i: (i, 0)),
                pl.BlockSpec((FB, A * D), lambda i: (i, 0)),
                pl.BlockSpec((1, D), lambda i: (0, 0)),
                pl.BlockSpec((1, D), lambda i: (0, 0)),
                pl.BlockSpec((1, D), lambda i: (0, 0)),
            ],
            out_specs=pl.BlockSpec((FB, A * D), lambda i: (i, 0)),
            out_shape=jax.ShapeDtypeStruct((F, A * D), f32),
        )(wm, xl.reshape(F, A * HD), xn.reshape(F, A * D),
          p[f'gat{li}_bias'][None, :], p[f'ln{li}_g'][None, :],
          p[f'ln{li}_b'][None, :]).reshape(M, D)

    return xn.reshape(B, T, A, D).transpose(0, 2, 1, 3).reshape(N, T, D)


# speedup vs baseline: 2.6501x; 1.0325x over previous
import functools

import jax
import jax.numpy as jnp
from jax import lax
from jax.experimental import pallas as pl
from jax.experimental.pallas import tpu as pltpu
from jax.experimental.pallas import tpu_sc as plsc

B, A, T, D, H = 32, 12, 80, 128, 8
E_RAW = 132
EP = 144
NG = EP // 16
N = B * A
F = B * T
M = F * A
HD = H * D
R_ROWS = N * T
BLK = 2048
NBLK = M // BLK
FB = 64
NFB = F // FB
NW = 32
FPW = F // NW
HDP = HD + 8
XLW = (A + 1) * HDP


def _mlp1_body(hexid_ref, pid_ref, th_ref, tp_ref, ti_ref, b1_ref,
               out_ref, stats_ref):
    i = pl.program_id(0)
    rows = hexid_ref.shape[0]
    hexid = hexid_ref[...]
    pid = pid_ref[...]
    ioh = lax.broadcasted_iota(jnp.int32, (rows, 600), 1)
    oh_h = (ioh == hexid).astype(jnp.float32)
    oh_p = (ioh == pid).astype(jnp.float32)
    grow = i * rows + lax.broadcasted_iota(jnp.int32, (rows, 1), 0)
    aid = (grow // T) % A
    oh_a = (lax.broadcasted_iota(jnp.int32, (rows, A), 1) == aid
            ).astype(jnp.float32)
    y = (jnp.dot(oh_h, th_ref[...], preferred_element_type=jnp.float32)
         + jnp.dot(oh_p, tp_ref[...], preferred_element_type=jnp.float32)
         + jnp.dot(oh_a, ti_ref[...], preferred_element_type=jnp.float32)
         + b1_ref[...])
    out_ref[...] = y

    @pl.when(i == 0)
    def _():
        stats_ref[...] = jnp.zeros_like(stats_ref)

    stats_ref[0:1, :] += jnp.sum(y, axis=0, keepdims=True)
    stats_ref[1:2, :] += jnp.sum(y * y, axis=0, keepdims=True)


def _mlp_body(y_ref, st_ref, g_ref, be_ref, w_ref, b_ref, out_ref, stats_ref):
    i = pl.program_id(0)
    mu = st_ref[0:1, :] / R_ROWS
    ms = st_ref[1:2, :] / R_ROWS
    inv = lax.rsqrt(ms - mu * mu + 1e-5)
    h = (y_ref[...] - mu) * inv * g_ref[...] + be_ref[...]
    h = jnp.maximum(h, 0.0)
    y = jnp.dot(h, w_ref[...], preferred_element_type=jnp.float32) + b_ref[...]
    out_ref[...] = y

    @pl.when(i == 0)
    def _():
        stats_ref[...] = jnp.zeros_like(stats_ref)

    stats_ref[0:1, :] += jnp.sum(y, axis=0, keepdims=True)
    stats_ref[1:2, :] += jnp.sum(y * y, axis=0, keepdims=True)


def _bnfin_body(y_ref, st_ref, g_ref, be_ref, out_ref):
    mu = st_ref[0:1, :] / R_ROWS
    ms = st_ref[1:2, :] / R_ROWS
    inv = lax.rsqrt(ms - mu * mu + 1e-5)
    h = (y_ref[...] - mu) * inv * g_ref[...] + be_ref[...]
    out_ref[...] = jnp.maximum(h, 0.0)


def _proj_body(x_ref, wl_ref, bl_ref, wr_ref, br_ref, xl_ref, xr_ref):
    x = x_ref[...]
    xl_ref[...] = jnp.dot(x, wl_ref[...],
                          preferred_element_type=jnp.float32) + bl_ref[...]
    xr_ref[...] = jnp.dot(x, wr_ref[...],
                          preferred_element_type=jnp.float32) + br_ref[...]


def _agg_body(wm_ref, xl_ref, xres_ref, bias_ref, g_ref, b_ref, out_ref):
    wm = wm_ref[...]
    xlb = xl_ref[...]
    for j in range(A):
        acc = jnp.zeros((FB, D), jnp.float32)
        for h in range(H):
            for i in range(A):
                w = wm[:, h * 256 + j * 16 + i][:, None]
                xs = xlb[:, i * HD + h * D:(i * HD + h * D) + D]
                acc = acc + w * xs
        v = acc * (1.0 / H) + bias_ref[...]
        v = v + xres_ref[:, j * D:(j + 1) * D]
        mu = jnp.mean(v, axis=-1, keepdims=True)
        var = jnp.mean((v - mu) * (v - mu), axis=-1, keepdims=True)
        v = (v - mu) * lax.rsqrt(var + 1e-5) * g_ref[...] + b_ref[...]
        out_ref[:, j * D:(j + 1) * D] = v


def _sc_body(xl_hbm, xr_hbm, ed_hbm, ea_hbm, wea_hbm, wout_hbm,
             xl_v, xr_v, ed_v, ea_v, wea_v, w_v, den_v):
    wid = lax.axis_index("s") * 2 + lax.axis_index("c")
    zi = jnp.zeros((16,), jnp.int32)
    zf = jnp.zeros((16,), jnp.float32)

    pltpu.sync_copy(wea_hbm, wea_v)
    for k in range(HD // 16):
        xl_v[pl.ds(A * HDP + k * 16, 16)] = zf
        xr_v[pl.ds(A * HDP + k * 16, 16)] = zf

    def frame_body(k, carry):
        f = wid * FPW + k
        pltpu.sync_copy(xl_hbm.at[pl.ds(f * (A * HDP), A * HDP)],
                        xl_v.at[pl.ds(0, A * HDP)])
        pltpu.sync_copy(xr_hbm.at[pl.ds(f * (A * HDP), A * HDP)],
                        xr_v.at[pl.ds(0, A * HDP)])
        pltpu.sync_copy(ed_hbm.at[f], ed_v)
        pltpu.sync_copy(ea_hbm.at[f], ea_v)

        def zero_body(q, c):
            w_v[pl.ds(q * 16, 16)] = zf
            return c

        lax.fori_loop(0, 128, zero_body, 0)
        for q in range(8):
            den_v[pl.ds(q * 16, 16)] = zf

        sb = [ed_v[pl.ds(g * 16, 16)] for g in range(NG)]
        db = [ed_v[pl.ds(EP + g * 16, 16)] for g in range(NG)]
        wx = [ed_v[pl.ds(2 * EP + g * 16, 16)] for g in range(NG)]
        dn = [ed_v[pl.ds(3 * EP + g * 16, 16)] for g in range(NG)]
        at = [ea_v[pl.ds(g * 16, 16)] for g in range(NG)]

        for h in range(H):
            hbase = h * D

            def dbody(d, accs):
                offv = zi + (hbase + d)
                we = wea_v[pl.ds((hbase + d) * 16, 16)]
                aw = wea_v[pl.ds(HD * 16 + (hbase + d) * 16, 16)]
                out = []
                for g in range(NG):
                    gl = plsc.load_gather(xl_v, [sb[g] + offv])
                    gr = plsc.load_gather(xr_v, [db[g] + offv])
                    z = gl + gr + at[g] * we
                    z = jnp.maximum(z, 0.2 * z)
                    out.append(accs[g] + z * aw)
                return tuple(out)

            accs = lax.fori_loop(0, D, dbody, tuple(zf for _ in range(NG)))

            m = accs[0]
            for g in range(1, NG):
                m = jnp.maximum(m, accs[g])
            mx = jnp.max(m)
            exs = [jnp.exp(a - mx) for a in accs]
            for g in range(NG):
                plsc.addupdate_scatter(den_v, [dn[g] + (h * 16)], exs[g])
            for g in range(NG):
                dsum = plsc.load_gather(den_v, [dn[g] + (h * 16)])
                alpha = exs[g] / (dsum + 1e-16)
                plsc.addupdate_scatter(w_v, [wx[g] + (h * 256)], alpha)

        pltpu.sync_copy(w_v, wout_hbm.at[f])
        return carry

    lax.fori_loop(0, FPW, frame_body, 0)


def _sc_call(xl_flat, xr_flat, edata, eattr, wea):
    mesh = plsc.VectorSubcoreMesh(core_axis_name="c", subcore_axis_name="s",
                                  num_cores=2, num_subcores=16)
    fn = pl.kernel(
        _sc_body,
        out_type=jax.ShapeDtypeStruct((F, 2048), jnp.float32),
        mesh=mesh,
        scratch_types=[
            pltpu.VMEM((XLW,), jnp.float32),
            pltpu.VMEM((XLW,), jnp.float32),
            pltpu.VMEM((4 * EP,), jnp.int32),
            pltpu.VMEM((EP,), jnp.float32),
            pltpu.VMEM((2 * HD * 16,), jnp.float32),
            pltpu.VMEM((2048,), jnp.float32),
            pltpu.VMEM((128,), jnp.float32),
        ],
        compiler_params=pltpu.CompilerParams(needs_layout_passes=False),
    )
    return fn(xl_flat, xr_flat, edata, eattr, wea)


def _stats_spec():
    return pl.BlockSpec((8, None), lambda i: (0, 0))


def kernel(state_feat, padding_mask, agent_ids, edge_index, edge_attr, params):
    p = params
    f32 = jnp.float32

    hex_id = state_feat[..., -1]
    hex_id = jnp.where(hex_id == 1e9, 598.0, hex_id).astype(jnp.int32)
    hexid = hex_id.reshape(R_ROWS, 1)
    pid = jnp.broadcast_to(jnp.clip(agent_ids, 0)[:, None],
                           (N, T)).reshape(R_ROWS, 1).astype(jnp.int32)
    th = p['hex_table'] @ p['W1'][0:12]
    tp = p['player_table'] @ p['W1'][12:24]
    ti = p['index_table'] @ p['W1'][24:36]
    b1 = p['b1'][None, :]

    grid1 = pl.pallas_call(
        _mlp1_body,
        grid=(NBLK,),
        in_specs=[
            pl.BlockSpec((BLK, 1), lambda i: (i, 0)),
            pl.BlockSpec((BLK, 1), lambda i: (i, 0)),
            pl.BlockSpec((600, 64), lambda i: (0, 0)),
            pl.BlockSpec((600, 64), lambda i: (0, 0)),
            pl.BlockSpec((A, 64), lambda i: (0, 0)),
            pl.BlockSpec((1, 64), lambda i: (0, 0)),
        ],
        out_specs=[
            pl.BlockSpec((BLK, 64), lambda i: (i, 0)),
            pl.BlockSpec((8, 64), lambda i: (0, 0)),
        ],
        out_shape=[
            jax.ShapeDtypeStruct((R_ROWS, 64), f32),
            jax.ShapeDtypeStruct((8, 64), f32),
        ],
    )
    y1, s1 = grid1(hexid, pid, th, tp, ti, b1)

    def mlp_layer(y, st, g, be, w, b, fout):
        fin = y.shape[1]
        call = pl.pallas_call(
            _mlp_body,
            grid=(NBLK,),
            in_specs=[
                pl.BlockSpec((BLK, fin), lambda i: (i, 0)),
                pl.BlockSpec((8, fin), lambda i: (0, 0)),
                pl.BlockSpec((1, fin), lambda i: (0, 0)),
                pl.BlockSpec((1, fin), lambda i: (0, 0)),
                pl.BlockSpec((fin, fout), lambda i: (0, 0)),
                pl.BlockSpec((1, fout), lambda i: (0, 0)),
            ],
            out_specs=[
                pl.BlockSpec((BLK, fout), lambda i: (i, 0)),
                pl.BlockSpec((8, fout), lambda i: (0, 0)),
            ],
            out_shape=[
                jax.ShapeDtypeStruct((R_ROWS, fout), f32),
                jax.ShapeDtypeStruct((8, fout), f32),
            ],
        )
        return call(y, st, g[None, :], be[None, :], w, b[None, :])

    y2, s2 = mlp_layer(y1, s1, p['g1'], p['be1'], p['W2'], p['b2'], 256)
    y3, s3 = mlp_layer(y2, s2, p['g2'], p['be2'], p['W3'], p['b3'], D)

    y3n = y3.reshape(B, A, T, D).transpose(0, 2, 1, 3).reshape(M, D)
    xn = pl.pallas_call(
        _bnfin_body,
        grid=(NBLK,),
        in_specs=[
            pl.BlockSpec((BLK, D), lambda i: (i, 0)),
            pl.BlockSpec((8, D), lambda i: (0, 0)),
            pl.BlockSpec((1, D), lambda i: (0, 0)),
            pl.BlockSpec((1, D), lambda i: (0, 0)),
        ],
        out_specs=pl.BlockSpec((BLK, D), lambda i: (i, 0)),
        out_shape=jax.ShapeDtypeStruct((M, D), f32),
    )(y3n, s3, p['g3'][None, :], p['be3'][None, :])

    src = edge_index[:, :, 0, :].reshape(F, E_RAW).astype(jnp.int32)
    dst = edge_index[:, :, 1, :].reshape(F, E_RAW).astype(jnp.int32)
    padi = jnp.full((F, EP - E_RAW), A, jnp.int32)
    src = jnp.concatenate([src, padi], axis=1)
    dst = jnp.concatenate([dst, padi], axis=1)
    attr = jnp.concatenate(
        [edge_attr.reshape(F, E_RAW),
         jnp.zeros((F, EP - E_RAW), f32)], axis=1)
    edata = jnp.concatenate(
        [src * HDP, dst * HDP, dst * 16 + src, dst], axis=1)

    for li in range(3):
        wl, bl = p[f'gat{li}_Wl'], p[f'gat{li}_bl']
        wr, br = p[f'gat{li}_Wr'], p[f'gat{li}_br']
        xl, xr = pl.pallas_call(
            _proj_body,
            grid=(NBLK,),
            in_specs=[
                pl.BlockSpec((BLK, D), lambda i: (i, 0)),
                pl.BlockSpec((D, HD), lambda i: (0, 0)),
                pl.BlockSpec((1, HD), lambda i: (0, 0)),
                pl.BlockSpec((D, HD), lambda i: (0, 0)),
                pl.BlockSpec((1, HD), lambda i: (0, 0)),
            ],
            out_specs=[
                pl.BlockSpec((BLK, HD), lambda i: (i, 0)),
                pl.BlockSpec((BLK, HD), lambda i: (i, 0)),
            ],
            out_shape=[
                jax.ShapeDtypeStruct((M, HD), f32),
                jax.ShapeDtypeStruct((M, HD), f32),
            ],
        )(xn, wl, bl[None, :], wr, br[None, :])

        wea = jnp.concatenate(
            [jnp.repeat(p[f'gat{li}_We'].reshape(HD), 16),
             jnp.repeat(p[f'gat{li}_att'].reshape(HD), 16)])
        xlp = jnp.pad(xl, ((0, 0), (0, HDP - HD))).reshape(M * HDP)
        xrp = jnp.pad(xr, ((0, 0), (0, HDP - HD))).reshape(M * HDP)
        wm = _sc_call(xlp, xrp, edata, attr, wea)

        xn = pl.pallas_call(
            _agg_body,
            grid=(NFB,),
            in_specs=[
                pl.BlockSpec((FB, 2048), lambda i: (i, 0)),
                pl.BlockSpec((FB, A * HD), lambda i: (i, 0)),
                pl.BlockSpec((FB, A * D), lambda i: (i, 0)),
                pl.BlockSpec((1, D), lambda i: (0, 0)),
                pl.BlockSpec((1, D), lambda i: (0, 0)),
                pl.BlockSpec((1, D), lambda i: (0, 0)),
            ],
            out_specs=pl.BlockSpec((FB, A * D), lambda i: (i, 0)),
            out_shape=jax.ShapeDtypeStruct((F, A * D), f32),
        )(wm, xl.reshape(F, A * HD), xn.reshape(F, A * D),
          p[f'gat{li}_bias'][None, :], p[f'ln{li}_g'][None, :],
          p[f'ln{li}_b'][None, :]).reshape(M, D)

    return xn.reshape(B, T, A, D).transpose(0, 2, 1, 3).reshape(N, T, D)


# speedup vs baseline: 2.7145x; 1.0243x over previous
import functools

import jax
import jax.numpy as jnp
from jax import lax
from jax.experimental import pallas as pl
from jax.experimental.pallas import tpu as pltpu
from jax.experimental.pallas import tpu_sc as plsc

B, A, T, D, H = 32, 12, 80, 128, 8
E_RAW = 132
EP = 144
NG = EP // 16
N = B * A
F = B * T
M = F * A
HD = H * D
R_ROWS = N * T
BLK = 2048
NBLK = M // BLK
FB = 64
NFB = F // FB
NW = 32
FPW = F // NW
HDP = HD + 8
XLW = (A + 1) * HDP


def _mlp1_body(hexid_ref, pid_ref, th_ref, tp_ref, ti_ref, b1_ref,
               out_ref, stats_ref):
    i = pl.program_id(0)
    rows = hexid_ref.shape[0]
    hexid = hexid_ref[...]
    pid = pid_ref[...]
    ioh = lax.broadcasted_iota(jnp.int32, (rows, 600), 1)
    oh_h = (ioh == hexid).astype(jnp.float32)
    oh_p = (ioh == pid).astype(jnp.float32)
    grow = i * rows + lax.broadcasted_iota(jnp.int32, (rows, 1), 0)
    aid = (grow // T) % A
    oh_a = (lax.broadcasted_iota(jnp.int32, (rows, A), 1) == aid
            ).astype(jnp.float32)
    y = (jnp.dot(oh_h, th_ref[...], preferred_element_type=jnp.float32)
         + jnp.dot(oh_p, tp_ref[...], preferred_element_type=jnp.float32)
         + jnp.dot(oh_a, ti_ref[...], preferred_element_type=jnp.float32)
         + b1_ref[...])
    out_ref[...] = y

    @pl.when(i == 0)
    def _():
        stats_ref[...] = jnp.zeros_like(stats_ref)

    stats_ref[0:1, :] += jnp.sum(y, axis=0, keepdims=True)
    stats_ref[1:2, :] += jnp.sum(y * y, axis=0, keepdims=True)


def _mlp_body(y_ref, st_ref, g_ref, be_ref, w_ref, b_ref, out_ref, stats_ref):
    i = pl.program_id(0)
    mu = st_ref[0:1, :] / R_ROWS
    ms = st_ref[1:2, :] / R_ROWS
    inv = lax.rsqrt(ms - mu * mu + 1e-5)
    h = (y_ref[...] - mu) * inv * g_ref[...] + be_ref[...]
    h = jnp.maximum(h, 0.0)
    y = jnp.dot(h, w_ref[...], preferred_element_type=jnp.float32) + b_ref[...]
    out_ref[...] = y

    @pl.when(i == 0)
    def _():
        stats_ref[...] = jnp.zeros_like(stats_ref)

    stats_ref[0:1, :] += jnp.sum(y, axis=0, keepdims=True)
    stats_ref[1:2, :] += jnp.sum(y * y, axis=0, keepdims=True)


def _bnfin_body(y_ref, st_ref, g_ref, be_ref, out_ref):
    mu = st_ref[0:1, :] / R_ROWS
    ms = st_ref[1:2, :] / R_ROWS
    inv = lax.rsqrt(ms - mu * mu + 1e-5)
    h = (y_ref[...] - mu) * inv * g_ref[...] + be_ref[...]
    out_ref[...] = jnp.maximum(h, 0.0)


def _proj_body(x_ref, wl_ref, bl_ref, wr_ref, br_ref, xl_ref, xr_ref):
    x = x_ref[...]
    xl_ref[...] = jnp.dot(x, wl_ref[...],
                          preferred_element_type=jnp.float32) + bl_ref[...]
    xr_ref[...] = jnp.dot(x, wr_ref[...],
                          preferred_element_type=jnp.float32) + br_ref[...]


def _agg_body(wm_ref, xl_ref, xres_ref, bias_ref, g_ref, b_ref, out_ref):
    wm = wm_ref[...]
    xlb = xl_ref[...]
    for j in range(A):
        acc = jnp.zeros((FB, D), jnp.float32)
        for h in range(H):
            for i in range(A):
                w = wm[:, h * 256 + j * 16 + i][:, None]
                xs = xlb[:, i * HD + h * D:(i * HD + h * D) + D]
                acc = acc + w * xs
        v = acc * (1.0 / H) + bias_ref[...]
        v = v + xres_ref[:, j * D:(j + 1) * D]
        mu = jnp.mean(v, axis=-1, keepdims=True)
        var = jnp.mean((v - mu) * (v - mu), axis=-1, keepdims=True)
        v = (v - mu) * lax.rsqrt(var + 1e-5) * g_ref[...] + b_ref[...]
        out_ref[:, j * D:(j + 1) * D] = v


def _sc_body(xl_hbm, xr_hbm, ed_hbm, ea_hbm, wea_hbm, wout_hbm,
             xl_v, xr_v, ed_v, ea_v, wea_v, w_v, den_v):
    wid = lax.axis_index("s") * 2 + lax.axis_index("c")
    zi = jnp.zeros((16,), jnp.int32)
    zf = jnp.zeros((16,), jnp.float32)

    pltpu.sync_copy(wea_hbm, wea_v)
    for k in range(HD // 16):
        xl_v[pl.ds(A * HDP + k * 16, 16)] = zf
        xr_v[pl.ds(A * HDP + k * 16, 16)] = zf

    def frame_body(k, carry):
        f = wid * FPW + k
        pltpu.sync_copy(xl_hbm.at[pl.ds(f * (A * HDP), A * HDP)],
                        xl_v.at[pl.ds(0, A * HDP)])
        pltpu.sync_copy(xr_hbm.at[pl.ds(f * (A * HDP), A * HDP)],
                        xr_v.at[pl.ds(0, A * HDP)])
        pltpu.sync_copy(ed_hbm.at[f], ed_v)
        pltpu.sync_copy(ea_hbm.at[f], ea_v)

        def zero_body(q, c):
            w_v[pl.ds(q * 16, 16)] = zf
            return c

        lax.fori_loop(0, 128, zero_body, 0)
        for q in range(8):
            den_v[pl.ds(q * 16, 16)] = zf

        sb = [ed_v[pl.ds(g * 16, 16)] for g in range(NG)]
        db = [ed_v[pl.ds(EP + g * 16, 16)] for g in range(NG)]
        wx = [ed_v[pl.ds(2 * EP + g * 16, 16)] for g in range(NG)]
        dn = [ed_v[pl.ds(3 * EP + g * 16, 16)] for g in range(NG)]
        at = [ea_v[pl.ds(g * 16, 16)] for g in range(NG)]

        for h in range(H):
            hbase = h * D

            def dbody(d, accs):
                offv = zi + (hbase + d)
                we = wea_v[pl.ds((hbase + d) * 16, 16)]
                aw = wea_v[pl.ds(HD * 16 + (hbase + d) * 16, 16)]
                out = []
                for g in range(NG):
                    gl = plsc.load_gather(xl_v, [sb[g] + offv])
                    gr = plsc.load_gather(xr_v, [db[g] + offv])
                    z = gl + gr + at[g] * we
                    z = jnp.maximum(z, 0.2 * z)
                    out.append(accs[g] + z * aw)
                return tuple(out)

            accs = plsc.parallel_loop(
                0, D, 1, unroll=2, carry=tuple(zf for _ in range(NG)))(dbody)

            m = accs[0]
            for g in range(1, NG):
                m = jnp.maximum(m, accs[g])
            mx = jnp.max(m)
            exs = [jnp.exp(a - mx) for a in accs]
            for g in range(NG):
                plsc.addupdate_scatter(den_v, [dn[g] + (h * 16)], exs[g])
            for g in range(NG):
                dsum = plsc.load_gather(den_v, [dn[g] + (h * 16)])
                alpha = exs[g] / (dsum + 1e-16)
                plsc.addupdate_scatter(w_v, [wx[g] + (h * 256)], alpha)

        pltpu.sync_copy(w_v, wout_hbm.at[f])
        return carry

    lax.fori_loop(0, FPW, frame_body, 0)


def _sc_call(xl_flat, xr_flat, edata, eattr, wea):
    mesh = plsc.VectorSubcoreMesh(core_axis_name="c", subcore_axis_name="s",
                                  num_cores=2, num_subcores=16)
    fn = pl.kernel(
        _sc_body,
        out_type=jax.ShapeDtypeStruct((F, 2048), jnp.float32),
        mesh=mesh,
        scratch_types=[
            pltpu.VMEM((XLW,), jnp.float32),
            pltpu.VMEM((XLW,), jnp.float32),
            pltpu.VMEM((4 * EP,), jnp.int32),
            pltpu.VMEM((EP,), jnp.float32),
            pltpu.VMEM((2 * HD * 16,), jnp.float32),
            pltpu.VMEM((2048,), jnp.float32),
            pltpu.VMEM((128,), jnp.float32),
        ],
        compiler_params=pltpu.CompilerParams(needs_layout_passes=False),
    )
    return fn(xl_flat, xr_flat, edata, eattr, wea)


def _stats_spec():
    return pl.BlockSpec((8, None), lambda i: (0, 0))


def kernel(state_feat, padding_mask, agent_ids, edge_index, edge_attr, params):
    p = params
    f32 = jnp.float32

    hex_id = state_feat[..., -1]
    hex_id = jnp.where(hex_id == 1e9, 598.0, hex_id).astype(jnp.int32)
    hexid = hex_id.reshape(R_ROWS, 1)
    pid = jnp.broadcast_to(jnp.clip(agent_ids, 0)[:, None],
                           (N, T)).reshape(R_ROWS, 1).astype(jnp.int32)
    th = p['hex_table'] @ p['W1'][0:12]
    tp = p['player_table'] @ p['W1'][12:24]
    ti = p['index_table'] @ p['W1'][24:36]
    b1 = p['b1'][None, :]

    grid1 = pl.pallas_call(
        _mlp1_body,
        grid=(NBLK,),
        in_specs=[
            pl.BlockSpec((BLK, 1), lambda i: (i, 0)),
            pl.BlockSpec((BLK, 1), lambda i: (i, 0)),
            pl.BlockSpec((600, 64), lambda i: (0, 0)),
            pl.BlockSpec((600, 64), lambda i: (0, 0)),
            pl.BlockSpec((A, 64), lambda i: (0, 0)),
            pl.BlockSpec((1, 64), lambda i: (0, 0)),
        ],
        out_specs=[
            pl.BlockSpec((BLK, 64), lambda i: (i, 0)),
            pl.BlockSpec((8, 64), lambda i: (0, 0)),
        ],
        out_shape=[
            jax.ShapeDtypeStruct((R_ROWS, 64), f32),
            jax.ShapeDtypeStruct((8, 64), f32),
        ],
    )
    y1, s1 = grid1(hexid, pid, th, tp, ti, b1)

    def mlp_layer(y, st, g, be, w, b, fout):
        fin = y.shape[1]
        call = pl.pallas_call(
            _mlp_body,
            grid=(NBLK,),
            in_specs=[
                pl.BlockSpec((BLK, fin), lambda i: (i, 0)),
                pl.BlockSpec((8, fin), lambda i: (0, 0)),
                pl.BlockSpec((1, fin), lambda i: (0, 0)),
                pl.BlockSpec((1, fin), lambda i: (0, 0)),
                pl.BlockSpec((fin, fout), lambda i: (0, 0)),
                pl.BlockSpec((1, fout), lambda i: (0, 0)),
            ],
            out_specs=[
                pl.BlockSpec((BLK, fout), lambda i: (i, 0)),
                pl.BlockSpec((8, fout), lambda i: (0, 0)),
            ],
            out_shape=[
                jax.ShapeDtypeStruct((R_ROWS, fout), f32),
                jax.ShapeDtypeStruct((8, fout), f32),
            ],
        )
        return call(y, st, g[None, :], be[None, :], w, b[None, :])

    y2, s2 = mlp_layer(y1, s1, p['g1'], p['be1'], p['W2'], p['b2'], 256)
    y3, s3 = mlp_layer(y2, s2, p['g2'], p['be2'], p['W3'], p['b3'], D)

    y3n = y3.reshape(B, A, T, D).transpose(0, 2, 1, 3).reshape(M, D)
    xn = pl.pallas_call(
        _bnfin_body,
        grid=(NBLK,),
        in_specs=[
            pl.BlockSpec((BLK, D), lambda i: (i, 0)),
            pl.BlockSpec((8, D), lambda i: (0, 0)),
            pl.BlockSpec((1, D), lambda i: (0, 0)),
            pl.BlockSpec((1, D), lambda i: (0, 0)),
        ],
        out_specs=pl.BlockSpec((BLK, D), lambda i: (i, 0)),
        out_shape=jax.ShapeDtypeStruct((M, D), f32),
    )(y3n, s3, p['g3'][None, :], p['be3'][None, :])

    src = edge_index[:, :, 0, :].reshape(F, E_RAW).astype(jnp.int32)
    dst = edge_index[:, :, 1, :].reshape(F, E_RAW).astype(jnp.int32)
    padi = jnp.full((F, EP - E_RAW), A, jnp.int32)
    src = jnp.concatenate([src, padi], axis=1)
    dst = jnp.concatenate([dst, padi], axis=1)
    attr = jnp.concatenate(
        [edge_attr.reshape(F, E_RAW),
         jnp.zeros((F, EP - E_RAW), f32)], axis=1)
    edata = jnp.concatenate(
        [src * HDP, dst * HDP, dst * 16 + src, dst], axis=1)

    for li in range(3):
        wl, bl = p[f'gat{li}_Wl'], p[f'gat{li}_bl']
        wr, br = p[f'gat{li}_Wr'], p[f'gat{li}_br']
        xl, xr = pl.pallas_call(
            _proj_body,
            grid=(NBLK,),
            in_specs=[
                pl.BlockSpec((BLK, D), lambda i: (i, 0)),
                pl.BlockSpec((D, HD), lambda i: (0, 0)),
                pl.BlockSpec((1, HD), lambda i: (0, 0)),
                pl.BlockSpec((D, HD), lambda i: (0, 0)),
                pl.BlockSpec((1, HD), lambda i: (0, 0)),
            ],
            out_specs=[
                pl.BlockSpec((BLK, HD), lambda i: (i, 0)),
                pl.BlockSpec((BLK, HD), lambda i: (i, 0)),
            ],
            out_shape=[
                jax.ShapeDtypeStruct((M, HD), f32),
                jax.ShapeDtypeStruct((M, HD), f32),
            ],
        )(xn, wl, bl[None, :], wr, br[None, :])

        wea = jnp.concatenate(
            [jnp.repeat(p[f'gat{li}_We'].reshape(HD), 16),
             jnp.repeat(p[f'gat{li}_att'].reshape(HD), 16)])
        xlp = jnp.pad(xl, ((0, 0), (0, HDP - HD))).reshape(M * HDP)
        xrp = jnp.pad(xr, ((0, 0), (0, HDP - HD))).reshape(M * HDP)
        wm = _sc_call(xlp, xrp, edata, attr, wea)

        xn = pl.pallas_call(
            _agg_body,
            grid=(NFB,),
            in_specs=[
                pl.BlockSpec((FB, 2048), lambda i: (i, 0)),
                pl.BlockSpec((FB, A * HD), lambda i: (i, 0)),
                pl.BlockSpec((FB, A * D), lambda i: (i, 0)),
                pl.BlockSpec((1, D), lambda i: (0, 0)),
                pl.BlockSpec((1, D), lambda i: (0, 0)),
                pl.BlockSpec((1, D), lambda i: (0, 0)),
            ],
            out_specs=pl.BlockSpec((FB, A * D), lambda i: (i, 0)),
            out_shape=jax.ShapeDtypeStruct((F, A * D), f32),
        )(wm, xl.reshape(F, A * HD), xn.reshape(F, A * D),
          p[f'gat{li}_bias'][None, :], p[f'ln{li}_g'][None, :],
          p[f'ln{li}_b'][None, :]).reshape(M, D)

    return xn.reshape(B, T, A, D).transpose(0, 2, 1, 3).reshape(N, T, D)
